# baseline (speedup 1.0000x reference)
import functools

import jax
import jax.numpy as jnp
from jax import lax
from jax.experimental import pallas as pl
from jax.experimental.pallas import tpu as pltpu
from jax.experimental.pallas import tpu_sc as plsc

_CHUNK = 40
_NJOB = 5
_D = 128
_K = 3 * _D

_mesh = plsc.VectorSubcoreMesh(core_axis_name="c", subcore_axis_name="s")


_TPM = 5


@functools.partial(
    pl.kernel,
    mesh=_mesh,
    out_type=jax.ShapeDtypeStruct((_NJOB, _K), jnp.float32),
    scratch_types=[
        pltpu.VMEM((_CHUNK,), jnp.int32),
        pltpu.VMEM((_CHUNK, _D), jnp.float32),
        pltpu.VMEM((_D,), jnp.float32),
        pltpu.SemaphoreType.DMA,
    ],
)
def _gather_pool(pat_hbm, e0, e1, e2, out_hbm, idx_v, rows_v, acc_v, sem_g):
    wid = lax.axis_index("s") * 2 + lax.axis_index("c")

    @pl.when(wid < 3 * _TPM)
    def _():
        m = wid // _TPM
        p = wid % _TPM
        off = 600 + 200 * m - 600 * (m // 2) + _CHUNK * p
        pltpu.sync_copy(pat_hbm.at[pl.ds(pl.multiple_of(off, 8), _CHUNK)],
                        idx_v)

        @pl.when(m == 0)
        def _():
            pltpu.async_copy(e0.at[idx_v], rows_v, sem_g).wait()

        @pl.when(m == 1)
        def _():
            pltpu.async_copy(e1.at[idx_v], rows_v, sem_g).wait()

        @pl.when(m == 2)
        def _():
            pltpu.async_copy(e2.at[idx_v], rows_v, sem_g).wait()

        for v in range(_D // 16):
            a = rows_v[0, pl.ds(v * 16, 16)]
            for r in range(1, _CHUNK):
                a = a + rows_v[r, pl.ds(v * 16, 16)]
            acc_v[pl.ds(v * 16, 16)] = a
        pltpu.sync_copy(acc_v, out_hbm.at[p, pl.ds(pl.multiple_of(m * _D, 8),
                                                   _D)])


_NB = 8
_BO = 128


def _dense(partial_ref, w_ref, b_ref, ddi_ref, res_ref, bn_ref, ssum_ref):
    i = pl.program_id(0)
    rep = jnp.sum(partial_ref[:], axis=0, keepdims=True)
    rep = jnp.maximum(rep, 0.0)
    out = lax.dot_general(
        rep, w_ref[:],
        dimension_numbers=(((1,), (1,)), ((), ())),
        preferred_element_type=jnp.float32,
    ) + b_ref[:]
    res_ref[:] = out
    neg = jax.nn.sigmoid(out)
    glob = i * _BO + lax.broadcasted_iota(jnp.int32, (1, _BO), 1)
    neg = jnp.where(glob < 1000, neg, 0.0)

    @pl.when(i == 0)
    def _():
        ssum_ref[0] = 0.0

    ssum_ref[0] += jnp.sum(neg)

    @pl.when(i == _NB - 1)
    def _():
        s = ssum_ref[0]
        bn_ref[:] = jnp.reshape(0.0005 * ddi_ref[0, 0] * s * s, (1, 1))


def kernel(patient, E0, E1, E2, W, b, ddi_adj):
    partial = _gather_pool(patient.reshape(-1), E0, E1, E2)
    result, bn = pl.pallas_call(
        _dense,
        grid=(_NB,),
        in_specs=[
            pl.BlockSpec((_NJOB, _K), lambda i: (0, 0)),
            pl.BlockSpec((_BO, _K), lambda i: (i, 0)),
            pl.BlockSpec((1, _BO), lambda i: (0, i)),
            pl.BlockSpec((1, 1), lambda i: (0, 0)),
        ],
        out_specs=(
            pl.BlockSpec((1, _BO), lambda i: (0, i)),
            pl.BlockSpec((1, 1), lambda i: (0, 0)),
        ),
        out_shape=(
            jax.ShapeDtypeStruct((1, 1000), jnp.float32),
            jax.ShapeDtypeStruct((1, 1), jnp.float32),
        ),
        scratch_shapes=[pltpu.SMEM((1,), jnp.float32)],
    )(partial, W, b.reshape(1, 1000), ddi_adj)
    return (result, bn.reshape(()))


# speedup vs baseline: 1.1313x; 1.1313x over previous
import functools

import jax
import jax.numpy as jnp
from jax import lax
from jax.experimental import pallas as pl
from jax.experimental.pallas import tpu as pltpu
from jax.experimental.pallas import tpu_sc as plsc

_CHUNK = 40
_NJOB = 5
_D = 128
_K = 3 * _D

_mesh = plsc.VectorSubcoreMesh(core_axis_name="c", subcore_axis_name="s")


_TPM = 5


@functools.partial(
    pl.kernel,
    mesh=_mesh,
    out_type=jax.ShapeDtypeStruct((_NJOB, _K), jnp.float32),
    scratch_types=[
        pltpu.VMEM((_CHUNK,), jnp.int32),
        pltpu.VMEM((_CHUNK, _D), jnp.float32),
        pltpu.VMEM((_D,), jnp.float32),
        pltpu.SemaphoreType.DMA,
    ],
)
def _gather_pool(p0_hbm, p1_hbm, p2_hbm, e0, e1, e2, out_hbm,
                 idx_v, rows_v, acc_v, sem_g):
    wid = lax.axis_index("s") * 2 + lax.axis_index("c")

    @pl.when(wid < 3 * _TPM)
    def _():
        m = wid // _TPM
        p = wid % _TPM
        off = pl.multiple_of(_CHUNK * p, 8)

        @pl.when(m == 0)
        def _():
            pltpu.sync_copy(p0_hbm.at[pl.ds(off, _CHUNK)], idx_v)
            pltpu.async_copy(e0.at[idx_v], rows_v, sem_g).wait()

        @pl.when(m == 1)
        def _():
            pltpu.sync_copy(p1_hbm.at[pl.ds(off, _CHUNK)], idx_v)
            pltpu.async_copy(e1.at[idx_v], rows_v, sem_g).wait()

        @pl.when(m == 2)
        def _():
            pltpu.sync_copy(p2_hbm.at[pl.ds(off, _CHUNK)], idx_v)
            pltpu.async_copy(e2.at[idx_v], rows_v, sem_g).wait()

        for v in range(_D // 16):
            a = rows_v[0, pl.ds(v * 16, 16)]
            for r in range(1, _CHUNK):
                a = a + rows_v[r, pl.ds(v * 16, 16)]
            acc_v[pl.ds(v * 16, 16)] = a
        pltpu.sync_copy(acc_v, out_hbm.at[p, pl.ds(pl.multiple_of(m * _D, 8),
                                                   _D)])


def _dense(partial_ref, w_ref, b_ref, ddi_ref, res_ref, bn_ref):
    rep = jnp.sum(partial_ref[:], axis=0, keepdims=True)
    rep = jnp.maximum(rep, 0.0)
    out = lax.dot_general(
        rep, w_ref[:],
        dimension_numbers=(((1,), (1,)), ((), ())),
        preferred_element_type=jnp.float32,
    ) + b_ref[:]
    res_ref[:] = out
    neg = jax.nn.sigmoid(out)
    s = jnp.sum(neg)
    bn_ref[:] = jnp.reshape(0.0005 * ddi_ref[0, 0] * s * s, (1, 1))


def kernel(patient, E0, E1, E2, W, b, ddi_adj):
    partial = _gather_pool(patient[1, 0], patient[1, 1], patient[0, 2],
                           E0, E1, E2)
    result, bn = pl.pallas_call(
        _dense,
        out_shape=(
            jax.ShapeDtypeStruct((1, 1000), jnp.float32),
            jax.ShapeDtypeStruct((1, 1), jnp.float32),
        ),
    )(partial, W, b.reshape(1, 1000), ddi_adj)
    return (result, bn.reshape(()))


# speedup vs baseline: 1.1488x; 1.0155x over previous
import functools

import jax
import jax.numpy as jnp
from jax import lax
from jax.experimental import pallas as pl
from jax.experimental.pallas import tpu as pltpu
from jax.experimental.pallas import tpu_sc as plsc

_CHUNK = 40
_NJOB = 5
_D = 128
_K = 3 * _D

_mesh = plsc.VectorSubcoreMesh(core_axis_name="c", subcore_axis_name="s")


_TPM = 5


@functools.partial(
    pl.kernel,
    mesh=_mesh,
    out_type=jax.ShapeDtypeStruct((_NJOB, _K), jnp.float32),
    scratch_types=[
        pltpu.VMEM((_CHUNK,), jnp.int32),
        pltpu.VMEM((_CHUNK, _D), jnp.float32),
        pltpu.VMEM((_D,), jnp.float32),
        pltpu.SemaphoreType.DMA,
    ],
)
def _gather_pool(pat_hbm, e0, e1, e2, out_hbm, idx_v, rows_v, acc_v, sem_g):
    wid = lax.axis_index("s") * 2 + lax.axis_index("c")

    @pl.when(wid < 3 * _TPM)
    def _():
        m = wid // _TPM
        p = wid % _TPM
        off = 600 + 200 * m - 600 * (m // 2) + _CHUNK * p
        pltpu.sync_copy(pat_hbm.at[pl.ds(pl.multiple_of(off, 8), _CHUNK)],
                        idx_v)

        @pl.when(m == 0)
        def _():
            pltpu.async_copy(e0.at[idx_v], rows_v, sem_g).wait()

        @pl.when(m == 1)
        def _():
            pltpu.async_copy(e1.at[idx_v], rows_v, sem_g).wait()

        @pl.when(m == 2)
        def _():
            pltpu.async_copy(e2.at[idx_v], rows_v, sem_g).wait()

        for v in range(_D // 16):
            a = rows_v[0, pl.ds(v * 16, 16)]
            for r in range(1, _CHUNK):
                a = a + rows_v[r, pl.ds(v * 16, 16)]
            acc_v[pl.ds(v * 16, 16)] = a
        pltpu.sync_copy(acc_v, out_hbm.at[p, pl.ds(pl.multiple_of(m * _D, 8),
                                                   _D)])


def _dense(partial_ref, w_ref, b_ref, ddi_ref, res_ref, bn_ref):
    rep = jnp.sum(partial_ref[:], axis=0, keepdims=True)
    rep = jnp.maximum(rep, 0.0)
    out = lax.dot_general(
        rep, w_ref[:],
        dimension_numbers=(((1,), (1,)), ((), ())),
        preferred_element_type=jnp.float32,
    ) + b_ref[:][None, :]
    res_ref[:] = out
    neg = jax.nn.sigmoid(out)
    s = jnp.sum(neg)
    bn_ref[:] = jnp.reshape(0.0005 * ddi_ref[0, 0] * s * s, (1, 1))


def kernel(patient, E0, E1, E2, W, b, ddi_adj):
    partial = _gather_pool(patient.reshape(-1), E0, E1, E2)
    result, bn = pl.pallas_call(
        _dense,
        out_shape=(
            jax.ShapeDtypeStruct((1, 1000), jnp.float32),
            jax.ShapeDtypeStruct((1, 1), jnp.float32),
        ),
    )(partial, W, b, ddi_adj)
    return (result, bn.reshape(()))
